# baseline (speedup 1.0000x reference)
import functools

import jax
import jax.numpy as jnp
from jax.experimental import pallas as pl
from jax.experimental.pallas import tpu as pltpu

QUARTER = 192
SEQ = 81
BB = 16


def _enc_from_tables(row, col, box, pos):
    row81 = jnp.broadcast_to(row[:, None, :], (9, 9, QUARTER)).reshape(SEQ, QUARTER)
    col81 = jnp.broadcast_to(col[None, :, :], (9, 9, QUARTER)).reshape(SEQ, QUARTER)
    boxr = box.reshape(3, 3, QUARTER)
    box81 = jnp.broadcast_to(
        boxr[:, None, :, None, :], (3, 3, 3, 3, QUARTER)
    ).reshape(SEQ, QUARTER)
    return jnp.concatenate([row81, col81, box81, pos], axis=-1)


def _bcast_kernel(row_ref, col_ref, box_ref, pos_ref, out_ref, enc_ref):
    @pl.when(pl.program_id(0) == 0)
    def _():
        enc_ref[:] = _enc_from_tables(
            row_ref[:], col_ref[:], box_ref[:], pos_ref[:]
        )

    out_ref[:] = jnp.broadcast_to(enc_ref[:][None], out_ref.shape)


@functools.partial(jax.jit, static_argnames=("batch",))
def _run(row_embed, col_embed, box_embed, pos_embed, batch):
    grid = (batch // BB,)
    return pl.pallas_call(
        _bcast_kernel,
        grid=grid,
        in_specs=[
            pl.BlockSpec((9, QUARTER), lambda i: (0, 0)),
            pl.BlockSpec((9, QUARTER), lambda i: (0, 0)),
            pl.BlockSpec((9, QUARTER), lambda i: (0, 0)),
            pl.BlockSpec((SEQ, QUARTER), lambda i: (0, 0)),
        ],
        out_specs=pl.BlockSpec((BB, SEQ, 4 * QUARTER), lambda i: (i, 0, 0)),
        out_shape=jax.ShapeDtypeStruct((batch, SEQ, 4 * QUARTER), jnp.float32),
        scratch_shapes=[pltpu.VMEM((SEQ, 4 * QUARTER), jnp.float32)],
    )(row_embed, col_embed, box_embed, pos_embed)


def kernel(x, row_embed, col_embed, box_embed, pos_embed):
    batch = x.shape[0]
    return _run(row_embed, col_embed, box_embed, pos_embed, batch)


# speedup vs baseline: 1.0062x; 1.0062x over previous
import functools

import jax
import jax.numpy as jnp
from jax.experimental import pallas as pl
from jax.experimental.pallas import tpu as pltpu

QUARTER = 192
SEQ = 81
BB = 16
NSEM = 8


def _enc_from_tables(row, col, box, pos):
    row81 = jnp.broadcast_to(row[:, None, :], (9, 9, QUARTER)).reshape(SEQ, QUARTER)
    col81 = jnp.broadcast_to(col[None, :, :], (9, 9, QUARTER)).reshape(SEQ, QUARTER)
    boxr = box.reshape(3, 3, QUARTER)
    box81 = jnp.broadcast_to(
        boxr[:, None, :, None, :], (3, 3, 3, 3, QUARTER)
    ).reshape(SEQ, QUARTER)
    return jnp.concatenate([row81, col81, box81, pos], axis=-1)


def _bcast_kernel(nchunk, row_ref, col_ref, box_ref, pos_ref, out_ref,
                  blk_ref, sems):
    enc = _enc_from_tables(row_ref[:], col_ref[:], box_ref[:], pos_ref[:])
    blk_ref[:] = jnp.broadcast_to(enc[None], blk_ref.shape)

    def copy(i):
        return pltpu.make_async_copy(
            blk_ref, out_ref.at[pl.ds(i * BB, BB)], sems.at[i % NSEM]
        )

    for i in range(nchunk):
        if i >= NSEM:
            copy(i - NSEM).wait()
        copy(i).start()
    for i in range(max(nchunk - NSEM, 0), nchunk):
        copy(i).wait()


@functools.partial(jax.jit, static_argnames=("batch",))
def _run(row_embed, col_embed, box_embed, pos_embed, batch):
    nchunk = batch // BB
    assert nchunk * BB == batch
    return pl.pallas_call(
        functools.partial(_bcast_kernel, nchunk),
        in_specs=[
            pl.BlockSpec((9, QUARTER), lambda: (0, 0)),
            pl.BlockSpec((9, QUARTER), lambda: (0, 0)),
            pl.BlockSpec((9, QUARTER), lambda: (0, 0)),
            pl.BlockSpec((SEQ, QUARTER), lambda: (0, 0)),
        ],
        out_specs=pl.BlockSpec(memory_space=pltpu.MemorySpace.HBM),
        out_shape=jax.ShapeDtypeStruct((batch, SEQ, 4 * QUARTER), jnp.float32),
        scratch_shapes=[
            pltpu.VMEM((BB, SEQ, 4 * QUARTER), jnp.float32),
            pltpu.SemaphoreType.DMA((NSEM,)),
        ],
    )(row_embed, col_embed, box_embed, pos_embed)


def kernel(x, row_embed, col_embed, box_embed, pos_embed):
    batch = x.shape[0]
    return _run(row_embed, col_embed, box_embed, pos_embed, batch)
